# Initial kernel scaffold; baseline (speedup 1.0000x reference)
import jax
import jax.numpy as jnp
from jax.experimental import pallas as pl


def kernel(x, window_w, window_b, window_gamma, offset_w, offset_b, offset_gamma, kernel_w, kernel_b, kernel_gamma, v_w, v_b, out_w):
    raise NotImplementedError("write your pallas kernel here")


# speedup vs baseline: 44.3542x; 44.3542x over previous
import functools
import math

import jax
import jax.numpy as jnp
from jax import lax
from jax.experimental import pallas as pl
from jax.experimental.pallas import tpu as pltpu
from jax.experimental.pallas import tpu_sc as plsc

L = 2048
C = 768
H = 12
K = 64
D = C // H
MAX_WINDOW = float(min(int(math.sqrt(L)), K))
HALF = int(MAX_WINDOW) // 2
MAX_OFFSET = float(int(math.sqrt(L)))
NO = 2 * HALF + 1
NJ = NO + 1
MIN_WINDOW = 1.0

LB = 256
CHUNK = 64
ROWS = CHUNK + 2 * (HALF + MAX_OFFSET.__int__()) + 2
ROWS = 200
NTASK_L = L // CHUNK
NTASKS = NTASK_L * H
NWORKERS = 32
TASKS_PER_W = NTASKS // NWORKERS


def _rms_scale(x32):
    var = jnp.mean(x32 * x32, axis=-1, keepdims=True)
    return lax.rsqrt(var + 1e-6)


def _stage_a_body(x_ref, ww_ref, wb_ref, wg_ref, ow_ref, ob_ref, og_ref,
                  kw_ref, kb_ref, kg_ref, vw_ref, vb_ref,
                  ws_ref, co_ref, kt_ref, v_ref):
    xb = x_ref[...]
    wpre = jnp.dot(xb, ww_ref[...], preferred_element_type=jnp.float32) + wb_ref[...]
    wn = wg_ref[...] * (wpre * _rms_scale(wpre))
    wraw = 1.0 / (1.0 + jnp.exp(-wn))
    ws_ref[...] = MIN_WINDOW + wraw * (MAX_WINDOW - MIN_WINDOW)
    opre = jnp.dot(xb, ow_ref[...], preferred_element_type=jnp.float32) + ob_ref[...]
    on = og_ref[...] * (opre * _rms_scale(opre))
    co_ref[...] = jnp.tanh(on) * MAX_OFFSET
    kpre = jnp.dot(xb, kw_ref[...], preferred_element_type=jnp.float32) + kb_ref[...]
    kn = kg_ref[...] * (kpre * _rms_scale(kpre))
    kt_ref[...] = kn * (1.0 / (1.0 + jnp.exp(-kn)))
    v_ref[...] = jnp.dot(xb, vw_ref[...], preferred_element_type=jnp.float32) + vb_ref[...]


def _stage_a(x2, window_w, window_b, window_gamma, offset_w, offset_b,
             offset_gamma, kernel_w, kernel_b, kernel_gamma, v_w, v_b):
    nblk = L // LB
    full = lambda shape: pl.BlockSpec(shape, lambda i: tuple(0 for _ in shape))
    row = pl.BlockSpec((LB, C), lambda i: (i, 0))
    return pl.pallas_call(
        _stage_a_body,
        grid=(nblk,),
        in_specs=[
            row,
            full((C, H)), full((H,)), full((H,)),
            full((C, H)), full((H,)), full((H,)),
            full((C, H * K)), full((H * K,)), full((H * K,)),
            full((C, C)), full((C,)),
        ],
        out_specs=[
            pl.BlockSpec((LB, H), lambda i: (i, 0)),
            pl.BlockSpec((LB, H), lambda i: (i, 0)),
            pl.BlockSpec((LB, H * K), lambda i: (i, 0)),
            pl.BlockSpec((LB, C), lambda i: (i, 0)),
        ],
        out_shape=[
            jax.ShapeDtypeStruct((L, H), jnp.float32),
            jax.ShapeDtypeStruct((L, H), jnp.float32),
            jax.ShapeDtypeStruct((L, H * K), jnp.float32),
            jax.ShapeDtypeStruct((L, C), jnp.float32),
        ],
    )(x2, window_w, window_b, window_gamma, offset_w, offset_b, offset_gamma,
      kernel_w, kernel_b, kernel_gamma, v_w, v_b)


def _sc_body(ws_hbm, co_hbm, kw_hbm, v_hbm, out_hbm,
             vrows, ktab, wsv, cov, a_scr, e_scr, row_scr, out_scr, sem):
    nc = 2
    wid = lax.axis_index("s") * nc + lax.axis_index("c")
    iota16 = lax.iota(jnp.int32, 16)
    zeros16 = jnp.zeros((16,), jnp.float32)

    for g in range(4):
        a_scr[0, pl.ds(g * 16, 16)] = zeros16
        a_scr[NO + 1, pl.ds(g * 16, 16)] = zeros16

    def task_body(t, carry):
        tid = wid * TASKS_PER_W + t
        h = tid // NTASK_L
        l0 = (tid % NTASK_L) * CHUNK
        r0 = jnp.maximum(0, jnp.minimum(l0 - (HALF + 46), L - ROWS))

        pltpu.sync_copy(v_hbm.at[pl.ds(r0, ROWS), pl.ds(h * D, D)], vrows)
        pltpu.sync_copy(kw_hbm.at[pl.ds(l0, CHUNK), pl.ds(h * K, K)], ktab)
        pltpu.sync_copy(ws_hbm.at[h, pl.ds(l0, CHUNK)], wsv)
        pltpu.sync_copy(co_hbm.at[h, pl.ds(l0, CHUNK)], cov)

        def group_body(g, carry2):
            lofs = g * 16 + iota16
            lvec_f = (l0 + lofs).astype(jnp.float32)
            c16 = cov[pl.ds(g * 16, 16)]
            w16 = wsv[pl.ds(g * 16, 16)]
            fc_t = c16.astype(jnp.int32)
            fc = jnp.where(c16 < fc_t.astype(jnp.float32), fc_t - 1, fc_t)
            fr = c16 - fc.astype(jnp.float32)
            base = l0 + lofs + fc - HALF
            inv_hw = 1.0 / (w16 * 0.5 + 1e-6)

            def tap_body(tp, denom):
                o_f = tp.astype(jnp.float32) - HALF
                npos = lvec_f + c16 + o_f
                valid = (npos >= 0.0) & (npos < float(L))
                rel = jnp.abs(o_f) * inv_hw
                sg = 1.0 / (1.0 + jnp.exp(5.0 * (rel - 1.0)))
                wm = jnp.where(valid, sg, 0.0)
                kidx = jnp.minimum(rel, 1.0) * float(K - 1)
                kf = jnp.minimum(kidx.astype(jnp.int32), K - 2)
                wcc = kidx - kf.astype(jnp.float32)
                kvf = plsc.load_gather(ktab, [lofs, kf])
                kvc = plsc.load_gather(ktab, [lofs, kf + 1])
                a = (kvf * (1.0 - wcc) + kvc * wcc) * wm
                a_scr[tp + 1, pl.ds(g * 16, 16)] = a
                return denom + a

            denom = lax.fori_loop(0, NO, tap_body, zeros16)
            inv_den = 1.0 / (denom + 1e-6)
            fr1 = (1.0 - fr) * inv_den
            fr2 = fr * inv_den

            def fold_body(j, carry3):
                a_j = a_scr[j + 1, pl.ds(g * 16, 16)]
                a_jm1 = a_scr[j, pl.ds(g * 16, 16)]
                e = fr1 * a_j + fr2 * a_jm1
                rowj = jnp.clip(base + j, 0, L - 1) - r0
                e_scr[j, pl.ds(g * 16, 16)] = e
                row_scr[j, pl.ds(g * 16, 16)] = rowj
                return carry3

            lax.fori_loop(0, NJ, fold_body, 0)
            return carry2

        lax.fori_loop(0, 4, group_body, 0)

        DC = 8
        for g in range(4):
            sl = pl.ds(g * 16, 16)
            for dc in range(D // DC):
                def j_body(j, accs, _g=g, _dc=dc, _sl=sl):
                    evec = e_scr[j, _sl]
                    rvec = row_scr[j, _sl]
                    new = []
                    for dd in range(DC):
                        dcol = jnp.full((16,), _dc * DC + dd, jnp.int32)
                        gv = plsc.load_gather(vrows, [rvec, dcol])
                        new.append(accs[dd] + evec * gv)
                    return tuple(new)

                accs = lax.fori_loop(0, NJ, j_body,
                                     tuple([zeros16] * DC))
                for dd in range(DC):
                    out_scr[dc * DC + dd, sl] = accs[dd]

        pltpu.sync_copy(out_scr, out_hbm.at[pl.ds(h * D, D), pl.ds(l0, CHUNK)])
        return carry

    lax.fori_loop(0, TASKS_PER_W, task_body, 0)


def _stage_b(ws_t, co_t, kw, v):
    mesh = plsc.VectorSubcoreMesh(core_axis_name="c", subcore_axis_name="s",
                                  num_cores=2, num_subcores=16)
    f = functools.partial(
        pl.kernel,
        out_type=jax.ShapeDtypeStruct((C, L), jnp.float32),
        mesh=mesh,
        scratch_types=[
            pltpu.VMEM((ROWS, D), jnp.float32),
            pltpu.VMEM((CHUNK, K), jnp.float32),
            pltpu.VMEM((CHUNK,), jnp.float32),
            pltpu.VMEM((CHUNK,), jnp.float32),
            pltpu.VMEM((NO + 2, CHUNK), jnp.float32),
            pltpu.VMEM((NJ, CHUNK), jnp.float32),
            pltpu.VMEM((NJ, CHUNK), jnp.int32),
            pltpu.VMEM((D, CHUNK), jnp.float32),
            pltpu.SemaphoreType.DMA,
        ],
        compiler_params=pltpu.CompilerParams(use_tc_tiling_on_sc=False,
                                             needs_layout_passes=False),
    )(_sc_body)
    return f(ws_t, co_t, kw, v)


def _stage_c_body(opt_ref, ow_ref, out_ref):
    y = lax.dot_general(opt_ref[...], ow_ref[...],
                        dimension_numbers=(((0,), (0,)), ((), ())),
                        preferred_element_type=jnp.float32)
    out_ref[...] = y * (1.0 / (1.0 + jnp.exp(-y)))


def _stage_c(op_t, out_w):
    nblk = L // LB
    return pl.pallas_call(
        _stage_c_body,
        grid=(nblk,),
        in_specs=[
            pl.BlockSpec((C, LB), lambda i: (0, i)),
            pl.BlockSpec((C, C), lambda i: (0, 0)),
        ],
        out_specs=pl.BlockSpec((LB, C), lambda i: (i, 0)),
        out_shape=jax.ShapeDtypeStruct((L, C), jnp.float32),
    )(op_t, out_w)


def kernel(x, window_w, window_b, window_gamma, offset_w, offset_b,
           offset_gamma, kernel_w, kernel_b, kernel_gamma, v_w, v_b, out_w):
    x2 = x[0]
    ws, co, kw, v = _stage_a(x2, window_w, window_b, window_gamma,
                             offset_w, offset_b, offset_gamma,
                             kernel_w, kernel_b, kernel_gamma, v_w, v_b)
    op_t = _stage_b(ws.T, co.T, kw, v)
    out = _stage_c(op_t, out_w)
    return out[None]


# speedup vs baseline: 130.4495x; 2.9411x over previous
import functools
import math

import jax
import jax.numpy as jnp
from jax import lax
from jax.experimental import pallas as pl
from jax.experimental.pallas import tpu as pltpu
from jax.experimental.pallas import tpu_sc as plsc

L = 2048
C = 768
H = 12
K = 64
D = C // H
MAX_WINDOW = float(min(int(math.sqrt(L)), K))
HALF = int(MAX_WINDOW) // 2
MAX_OFFSET = float(int(math.sqrt(L)))
NO = 2 * HALF + 1
NJ = NO + 1
MIN_WINDOW = 1.0

LB = 256
CHUNK = 64
ROWS = CHUNK + 2 * (HALF + MAX_OFFSET.__int__()) + 2
ROWS = 200
NTASK_L = L // CHUNK
NTASKS = NTASK_L * H
NWORKERS = 32
TASKS_PER_W = NTASKS // NWORKERS


def _rms_scale(x32):
    var = jnp.mean(x32 * x32, axis=-1, keepdims=True)
    return lax.rsqrt(var + 1e-6)


def _stage_a_body(x_ref, ww_ref, wb_ref, wg_ref, ow_ref, ob_ref, og_ref,
                  kw_ref, kb_ref, kg_ref, vw_ref, vb_ref,
                  ws_ref, co_ref, kt_ref, v_ref):
    xb = x_ref[...]
    wpre = jnp.dot(xb, ww_ref[...], preferred_element_type=jnp.float32) + wb_ref[...]
    wn = wg_ref[...] * (wpre * _rms_scale(wpre))
    wraw = 1.0 / (1.0 + jnp.exp(-wn))
    ws_ref[...] = MIN_WINDOW + wraw * (MAX_WINDOW - MIN_WINDOW)
    opre = jnp.dot(xb, ow_ref[...], preferred_element_type=jnp.float32) + ob_ref[...]
    on = og_ref[...] * (opre * _rms_scale(opre))
    co_ref[...] = jnp.tanh(on) * MAX_OFFSET
    kpre = jnp.dot(xb, kw_ref[...], preferred_element_type=jnp.float32) + kb_ref[...]
    kn = kg_ref[...] * (kpre * _rms_scale(kpre))
    kt_ref[...] = kn * (1.0 / (1.0 + jnp.exp(-kn)))
    v_ref[...] = jnp.dot(xb, vw_ref[...], preferred_element_type=jnp.float32) + vb_ref[...]


def _stage_a(x2, window_w, window_b, window_gamma, offset_w, offset_b,
             offset_gamma, kernel_w, kernel_b, kernel_gamma, v_w, v_b):
    nblk = L // LB
    full = lambda shape: pl.BlockSpec(shape, lambda i: tuple(0 for _ in shape))
    row = pl.BlockSpec((LB, C), lambda i: (i, 0))
    return pl.pallas_call(
        _stage_a_body,
        grid=(nblk,),
        in_specs=[
            row,
            full((C, H)), full((H,)), full((H,)),
            full((C, H)), full((H,)), full((H,)),
            full((C, H * K)), full((H * K,)), full((H * K,)),
            full((C, C)), full((C,)),
        ],
        out_specs=[
            pl.BlockSpec((LB, H), lambda i: (i, 0)),
            pl.BlockSpec((LB, H), lambda i: (i, 0)),
            pl.BlockSpec((LB, H * K), lambda i: (i, 0)),
            pl.BlockSpec((LB, C), lambda i: (i, 0)),
        ],
        out_shape=[
            jax.ShapeDtypeStruct((L, H), jnp.float32),
            jax.ShapeDtypeStruct((L, H), jnp.float32),
            jax.ShapeDtypeStruct((L, H * K), jnp.float32),
            jax.ShapeDtypeStruct((L, C), jnp.float32),
        ],
    )(x2, window_w, window_b, window_gamma, offset_w, offset_b, offset_gamma,
      kernel_w, kernel_b, kernel_gamma, v_w, v_b)


def _sc_body(ws_hbm, co_hbm, kw_hbm, v_hbm, out_hbm,
             vrows, ktab, wsv, cov, a_scr, e_scr, row_scr, out_scr, sem):
    nc = 2
    wid = lax.axis_index("s") * nc + lax.axis_index("c")
    iota16 = lax.iota(jnp.int32, 16)
    zeros16 = jnp.zeros((16,), jnp.float32)

    for g in range(4):
        a_scr[0, pl.ds(g * 16, 16)] = zeros16
        a_scr[NO + 1, pl.ds(g * 16, 16)] = zeros16

    def task_body(t, carry):
        tid = wid * TASKS_PER_W + t
        h = tid // NTASK_L
        l0 = (tid % NTASK_L) * CHUNK
        r0 = jnp.maximum(0, jnp.minimum(l0 - (HALF + 46), L - ROWS))

        pltpu.sync_copy(v_hbm.at[pl.ds(r0, ROWS), pl.ds(h * D, D)],
                        vrows.at[:, pl.ds(0, D)])
        pltpu.sync_copy(kw_hbm.at[pl.ds(l0, CHUNK), pl.ds(h * K, K)],
                        ktab.at[:, pl.ds(0, K)])
        pltpu.sync_copy(ws_hbm.at[h, pl.ds(l0, CHUNK)], wsv)
        pltpu.sync_copy(co_hbm.at[h, pl.ds(l0, CHUNK)], cov)

        def group_body(g, carry2):
            lofs = g * 16 + iota16
            lvec_f = (l0 + lofs).astype(jnp.float32)
            c16 = cov[pl.ds(g * 16, 16)]
            w16 = wsv[pl.ds(g * 16, 16)]
            fc_t = c16.astype(jnp.int32)
            fc = jnp.where(c16 < fc_t.astype(jnp.float32), fc_t - 1, fc_t)
            fr = c16 - fc.astype(jnp.float32)
            base = l0 + lofs + fc - HALF
            inv_hw = 1.0 / (w16 * 0.5 + 1e-6)

            def tap_body(tp, denom):
                o_f = tp.astype(jnp.float32) - HALF
                npos = lvec_f + c16 + o_f
                valid = (npos >= 0.0) & (npos < float(L))
                rel = jnp.abs(o_f) * inv_hw
                sg = 1.0 / (1.0 + jnp.exp(5.0 * (rel - 1.0)))
                wm = jnp.where(valid, sg, 0.0)
                kidx = jnp.minimum(rel, 1.0) * float(K - 1)
                kf = jnp.minimum(kidx.astype(jnp.int32), K - 2)
                wcc = kidx - kf.astype(jnp.float32)
                kvf = plsc.load_gather(ktab, [lofs, kf])
                kvc = plsc.load_gather(ktab, [lofs, kf + 1])
                a = (kvf * (1.0 - wcc) + kvc * wcc) * wm
                a_scr[tp + 1, pl.ds(g * 16, 16)] = a
                return denom + a

            denom = lax.fori_loop(0, NO, tap_body, zeros16)
            inv_den = 1.0 / (denom + 1e-6)
            fr1 = (1.0 - fr) * inv_den
            fr2 = fr * inv_den

            def fold_body(j, carry3):
                a_j = a_scr[j + 1, pl.ds(g * 16, 16)]
                a_jm1 = a_scr[j, pl.ds(g * 16, 16)]
                e = fr1 * a_j + fr2 * a_jm1
                rowj = jnp.clip(base + j, 0, L - 1) - r0
                e_scr[j, pl.ds(g * 16, 16)] = e
                row_scr[j, pl.ds(g * 16, 16)] = rowj
                return carry3

            lax.fori_loop(0, NJ, fold_body, 0)
            return carry2

        lax.fori_loop(0, 4, group_body, 0)

        DC = 8
        for g in range(4):
            sl = pl.ds(g * 16, 16)
            for dc in range(D // DC):
                def j_body(j, accs, _g=g, _dc=dc, _sl=sl):
                    evec = e_scr[j, _sl]
                    rvec = row_scr[j, _sl]
                    new = []
                    for dd in range(DC):
                        dcol = jnp.full((16,), _dc * DC + dd, jnp.int32)
                        gv = plsc.load_gather(vrows, [rvec, dcol])
                        new.append(accs[dd] + evec * gv)
                    return tuple(new)

                accs = lax.fori_loop(0, NJ, j_body,
                                     tuple([zeros16] * DC))
                for dd in range(DC):
                    out_scr[dc * DC + dd, sl] = accs[dd]

        pltpu.sync_copy(out_scr, out_hbm.at[pl.ds(h * D, D), pl.ds(l0, CHUNK)])
        return carry

    lax.fori_loop(0, TASKS_PER_W, task_body, 0)


def _stage_b(ws_t, co_t, kw, v):
    mesh = plsc.VectorSubcoreMesh(core_axis_name="c", subcore_axis_name="s",
                                  num_cores=2, num_subcores=16)
    f = functools.partial(
        pl.kernel,
        out_type=jax.ShapeDtypeStruct((C, L), jnp.float32),
        mesh=mesh,
        scratch_types=[
            pltpu.VMEM((ROWS, D + 1), jnp.float32),
            pltpu.VMEM((CHUNK, K + 1), jnp.float32),
            pltpu.VMEM((CHUNK,), jnp.float32),
            pltpu.VMEM((CHUNK,), jnp.float32),
            pltpu.VMEM((NO + 2, CHUNK), jnp.float32),
            pltpu.VMEM((NJ, CHUNK), jnp.float32),
            pltpu.VMEM((NJ, CHUNK), jnp.int32),
            pltpu.VMEM((D, CHUNK), jnp.float32),
            pltpu.SemaphoreType.DMA,
        ],
        compiler_params=pltpu.CompilerParams(use_tc_tiling_on_sc=False,
                                             needs_layout_passes=False),
    )(_sc_body)
    return f(ws_t, co_t, kw, v)


def _stage_c_body(opt_ref, ow_ref, out_ref):
    y = lax.dot_general(opt_ref[...], ow_ref[...],
                        dimension_numbers=(((0,), (0,)), ((), ())),
                        preferred_element_type=jnp.float32)
    out_ref[...] = y * (1.0 / (1.0 + jnp.exp(-y)))


def _stage_c(op_t, out_w):
    nblk = L // LB
    return pl.pallas_call(
        _stage_c_body,
        grid=(nblk,),
        in_specs=[
            pl.BlockSpec((C, LB), lambda i: (0, i)),
            pl.BlockSpec((C, C), lambda i: (0, 0)),
        ],
        out_specs=pl.BlockSpec((LB, C), lambda i: (i, 0)),
        out_shape=jax.ShapeDtypeStruct((L, C), jnp.float32),
    )(op_t, out_w)


def kernel(x, window_w, window_b, window_gamma, offset_w, offset_b,
           offset_gamma, kernel_w, kernel_b, kernel_gamma, v_w, v_b, out_w):
    x2 = x[0]
    ws, co, kw, v = _stage_a(x2, window_w, window_b, window_gamma,
                             offset_w, offset_b, offset_gamma,
                             kernel_w, kernel_b, kernel_gamma, v_w, v_b)
    op_t = _stage_b(ws.T, co.T, kw, v)
    out = _stage_c(op_t, out_w)
    return out[None]


# speedup vs baseline: 141.3836x; 1.0838x over previous
import functools
import math

import jax
import jax.numpy as jnp
from jax import lax
from jax.experimental import pallas as pl
from jax.experimental.pallas import tpu as pltpu
from jax.experimental.pallas import tpu_sc as plsc

L = 2048
C = 768
H = 12
K = 64
D = C // H
MAX_WINDOW = float(min(int(math.sqrt(L)), K))
HALF = int(MAX_WINDOW) // 2
MAX_OFFSET = float(int(math.sqrt(L)))
NO = 2 * HALF + 1
NJ = NO + 1
MIN_WINDOW = 1.0

LB = 256
CHUNK = 64
WPH = D // 2
ROWS = CHUNK + 2 * (HALF + MAX_OFFSET.__int__()) + 2
ROWS = 200
NTASK_L = L // CHUNK
NTASKS = NTASK_L * H
NWORKERS = 32
TASKS_PER_W = NTASKS // NWORKERS


def _rms_scale(x32):
    var = jnp.mean(x32 * x32, axis=-1, keepdims=True)
    return lax.rsqrt(var + 1e-6)


def _stage_a_body(x_ref, ww_ref, wb_ref, wg_ref, ow_ref, ob_ref, og_ref,
                  kw_ref, kb_ref, kg_ref, vw_ref, vb_ref,
                  ws_ref, co_ref, kt_ref, v_ref):
    xb = x_ref[...]
    wpre = jnp.dot(xb, ww_ref[...], preferred_element_type=jnp.float32) + wb_ref[...]
    wn = wg_ref[...] * (wpre * _rms_scale(wpre))
    wraw = 1.0 / (1.0 + jnp.exp(-wn))
    ws_ref[...] = MIN_WINDOW + wraw * (MAX_WINDOW - MIN_WINDOW)
    opre = jnp.dot(xb, ow_ref[...], preferred_element_type=jnp.float32) + ob_ref[...]
    on = og_ref[...] * (opre * _rms_scale(opre))
    co_ref[...] = jnp.tanh(on) * MAX_OFFSET
    kpre = jnp.dot(xb, kw_ref[...], preferred_element_type=jnp.float32) + kb_ref[...]
    kn = kg_ref[...] * (kpre * _rms_scale(kpre))
    kt_ref[...] = kn * (1.0 / (1.0 + jnp.exp(-kn)))
    v_ref[...] = jnp.dot(xb, vw_ref[...], preferred_element_type=jnp.float32) + vb_ref[...]


def _stage_a(x2, window_w, window_b, window_gamma, offset_w, offset_b,
             offset_gamma, kernel_w, kernel_b, kernel_gamma, v_w, v_b):
    nblk = L // LB
    full = lambda shape: pl.BlockSpec(shape, lambda i: tuple(0 for _ in shape))
    row = pl.BlockSpec((LB, C), lambda i: (i, 0))
    return pl.pallas_call(
        _stage_a_body,
        grid=(nblk,),
        in_specs=[
            row,
            full((C, H)), full((H,)), full((H,)),
            full((C, H)), full((H,)), full((H,)),
            full((C, H * K)), full((H * K,)), full((H * K,)),
            full((C, C)), full((C,)),
        ],
        out_specs=[
            pl.BlockSpec((LB, H), lambda i: (i, 0)),
            pl.BlockSpec((LB, H), lambda i: (i, 0)),
            pl.BlockSpec((LB, H * K), lambda i: (i, 0)),
            pl.BlockSpec((LB, C), lambda i: (i, 0)),
        ],
        out_shape=[
            jax.ShapeDtypeStruct((L, H), jnp.float32),
            jax.ShapeDtypeStruct((L, H), jnp.float32),
            jax.ShapeDtypeStruct((L, H * K), jnp.float32),
            jax.ShapeDtypeStruct((L, C), jnp.float32),
        ],
    )(x2, window_w, window_b, window_gamma, offset_w, offset_b, offset_gamma,
      kernel_w, kernel_b, kernel_gamma, v_w, v_b)


def _sc_body(ws_hbm, co_hbm, kw_hbm, v_hbm, out_hbm,
             vrows, ktab, wsv, cov, a_scr, e_scr, row_scr, out_scr, sem):
    nc = 2
    wid = lax.axis_index("s") * nc + lax.axis_index("c")
    iota16 = lax.iota(jnp.int32, 16)
    zeros16 = jnp.zeros((16,), jnp.float32)

    for g in range(4):
        a_scr[0, pl.ds(g * 16, 16)] = zeros16
        a_scr[NO + 1, pl.ds(g * 16, 16)] = zeros16

    def task_body(t, carry):
        tid = wid * TASKS_PER_W + t
        h = tid // NTASK_L
        l0 = (tid % NTASK_L) * CHUNK
        r0 = jnp.maximum(0, jnp.minimum(l0 - (HALF + 46), L - ROWS))

        pltpu.sync_copy(v_hbm.at[pl.ds(r0, ROWS), pl.ds(h * WPH, WPH)],
                        vrows.at[:, pl.ds(0, WPH)])
        pltpu.sync_copy(kw_hbm.at[pl.ds(l0, CHUNK), pl.ds(h * K, K)],
                        ktab.at[:, pl.ds(0, K)])
        pltpu.sync_copy(ws_hbm.at[h, pl.ds(l0, CHUNK)], wsv)
        pltpu.sync_copy(co_hbm.at[h, pl.ds(l0, CHUNK)], cov)

        def group_body(g, carry2):
            lofs = g * 16 + iota16
            lvec_f = (l0 + lofs).astype(jnp.float32)
            c16 = cov[pl.ds(g * 16, 16)]
            w16 = wsv[pl.ds(g * 16, 16)]
            fc_t = c16.astype(jnp.int32)
            fc = jnp.where(c16 < fc_t.astype(jnp.float32), fc_t - 1, fc_t)
            fr = c16 - fc.astype(jnp.float32)
            base = l0 + lofs + fc - HALF
            inv_hw = 1.0 / (w16 * 0.5 + 1e-6)

            def tap_body(tp, denom):
                o_f = tp.astype(jnp.float32) - HALF
                npos = lvec_f + c16 + o_f
                valid = (npos >= 0.0) & (npos < float(L))
                rel = jnp.abs(o_f) * inv_hw
                sg = 1.0 / (1.0 + jnp.exp(5.0 * (rel - 1.0)))
                wm = jnp.where(valid, sg, 0.0)
                kidx = jnp.minimum(rel, 1.0) * float(K - 1)
                kf = jnp.minimum(kidx.astype(jnp.int32), K - 2)
                wcc = kidx - kf.astype(jnp.float32)
                kvf = plsc.load_gather(ktab, [lofs, kf])
                kvc = plsc.load_gather(ktab, [lofs, kf + 1])
                a = (kvf * (1.0 - wcc) + kvc * wcc) * wm
                a_scr[tp + 1, pl.ds(g * 16, 16)] = a
                return denom + a

            denom = lax.fori_loop(0, NO, tap_body, zeros16)
            inv_den = 1.0 / (denom + 1e-6)
            fr1 = (1.0 - fr) * inv_den
            fr2 = fr * inv_den

            def fold_body(j, carry3):
                a_j = a_scr[j + 1, pl.ds(g * 16, 16)]
                a_jm1 = a_scr[j, pl.ds(g * 16, 16)]
                e = fr1 * a_j + fr2 * a_jm1
                rowj = jnp.clip(base + j, 0, L - 1) - r0
                e_scr[j, pl.ds(g * 16, 16)] = e
                row_scr[j, pl.ds(g * 16, 16)] = rowj
                return carry3

            lax.fori_loop(0, NJ, fold_body, 0)
            return carry2

        lax.fori_loop(0, 4, group_body, 0)

        WC = 8
        for g in range(4):
            sl = pl.ds(g * 16, 16)
            for wc in range(WPH // WC):
                def j_body(j, accs, _wc=wc, _sl=sl):
                    evec = e_scr[j, _sl]
                    rvec = row_scr[j, _sl]
                    new = []
                    for ww in range(WC):
                        wcol = jnp.full((16,), _wc * WC + ww, jnp.int32)
                        gv = plsc.load_gather(vrows, [rvec, wcol])
                        lo, hi = plsc.unpack(
                            plsc.bitcast(gv, jnp.bfloat16),
                            format=plsc.PackFormat.INTERLEAVED)
                        new.append(accs[2 * ww] + evec * lo)
                        new.append(accs[2 * ww + 1] + evec * hi)
                    return tuple(new)

                accs = lax.fori_loop(0, NJ, j_body,
                                     tuple([zeros16] * (2 * WC)))
                for ww in range(WC):
                    out_scr[2 * (wc * WC + ww), sl] = accs[2 * ww]
                    out_scr[2 * (wc * WC + ww) + 1, sl] = accs[2 * ww + 1]

        pltpu.sync_copy(out_scr, out_hbm.at[pl.ds(h * D, D), pl.ds(l0, CHUNK)])
        return carry

    lax.fori_loop(0, TASKS_PER_W, task_body, 0)


def _stage_b(ws_t, co_t, kw, v):
    mesh = plsc.VectorSubcoreMesh(core_axis_name="c", subcore_axis_name="s",
                                  num_cores=2, num_subcores=16)
    f = functools.partial(
        pl.kernel,
        out_type=jax.ShapeDtypeStruct((C, L), jnp.float32),
        mesh=mesh,
        scratch_types=[
            pltpu.VMEM((ROWS, WPH + 1), jnp.int32),
            pltpu.VMEM((CHUNK, K + 1), jnp.float32),
            pltpu.VMEM((CHUNK,), jnp.float32),
            pltpu.VMEM((CHUNK,), jnp.float32),
            pltpu.VMEM((NO + 2, CHUNK), jnp.float32),
            pltpu.VMEM((NJ, CHUNK), jnp.float32),
            pltpu.VMEM((NJ, CHUNK), jnp.int32),
            pltpu.VMEM((D, CHUNK), jnp.float32),
            pltpu.SemaphoreType.DMA,
        ],
        compiler_params=pltpu.CompilerParams(use_tc_tiling_on_sc=False,
                                             needs_layout_passes=False),
    )(_sc_body)
    return f(ws_t, co_t, kw, v)


def _stage_c_body(opt_ref, ow_ref, out_ref):
    y = lax.dot_general(opt_ref[...], ow_ref[...],
                        dimension_numbers=(((0,), (0,)), ((), ())),
                        preferred_element_type=jnp.float32)
    out_ref[...] = y * (1.0 / (1.0 + jnp.exp(-y)))


def _stage_c(op_t, out_w):
    nblk = L // LB
    return pl.pallas_call(
        _stage_c_body,
        grid=(nblk,),
        in_specs=[
            pl.BlockSpec((C, LB), lambda i: (0, i)),
            pl.BlockSpec((C, C), lambda i: (0, 0)),
        ],
        out_specs=pl.BlockSpec((LB, C), lambda i: (i, 0)),
        out_shape=jax.ShapeDtypeStruct((L, C), jnp.float32),
    )(op_t, out_w)


def kernel(x, window_w, window_b, window_gamma, offset_w, offset_b,
           offset_gamma, kernel_w, kernel_b, kernel_gamma, v_w, v_b, out_w):
    x2 = x[0]
    ws, co, kw, v = _stage_a(x2, window_w, window_b, window_gamma,
                             offset_w, offset_b, offset_gamma,
                             kernel_w, kernel_b, kernel_gamma, v_w, v_b)
    v_packed = lax.bitcast_convert_type(
        v.astype(jnp.bfloat16).reshape(L, C // 2, 2), jnp.int32)
    op_t = _stage_b(ws.T, co.T, kw, v_packed)
    out = _stage_c(op_t, out_w)
    return out[None]


# speedup vs baseline: 160.0452x; 1.1320x over previous
import functools
import math

import jax
import jax.numpy as jnp
from jax import lax
from jax.experimental import pallas as pl
from jax.experimental.pallas import tpu as pltpu
from jax.experimental.pallas import tpu_sc as plsc

L = 2048
C = 768
H = 12
K = 64
D = C // H
MAX_WINDOW = float(min(int(math.sqrt(L)), K))
HALF = int(MAX_WINDOW) // 2
MAX_OFFSET = float(int(math.sqrt(L)))
NO = 2 * HALF + 1
NJ = NO + 1
MIN_WINDOW = 1.0

LB = 256
CHUNK = 64
WPH = D // 2
ROWS = 200
NTASK_L = L // CHUNK
NTASKS = NTASK_L * H
NWORKERS = 32
TASKS_PER_W = NTASKS // NWORKERS


def _stage_a_body(x_ref, ww_ref, wb_ref, wg_ref, ow_ref, ob_ref, og_ref,
                  kw_ref, kb_ref, kg_ref, vwlo_ref, vblo_ref, vwhi_ref,
                  vbhi_ref, wst_ref, cot_ref, kt_ref, vp_ref):
    xb = x_ref[...]

    def proj_t(w_ref, b_ref):
        p = lax.dot_general(w_ref[...], xb, (((0,), (1,)), ((), ())),
                            preferred_element_type=jnp.float32)
        return p + b_ref[...][:, None]

    def rms_t(p, g_ref):
        var = jnp.mean(p * p, axis=0, keepdims=True)
        return g_ref[...][:, None] * (p * lax.rsqrt(var + 1e-6))

    wn = rms_t(proj_t(ww_ref, wb_ref), wg_ref)
    wst_ref[...] = MIN_WINDOW + (MAX_WINDOW - MIN_WINDOW) / (1.0 + jnp.exp(-wn))
    on = rms_t(proj_t(ow_ref, ob_ref), og_ref)
    cot_ref[...] = jnp.tanh(on) * MAX_OFFSET

    kpre = jnp.dot(xb, kw_ref[...], preferred_element_type=jnp.float32) \
        + kb_ref[...]
    kvar = jnp.mean(kpre * kpre, axis=-1, keepdims=True)
    kn = kg_ref[...] * (kpre * lax.rsqrt(kvar + 1e-6))
    kt_ref[...] = kn * (1.0 / (1.0 + jnp.exp(-kn)))

    vlo = jnp.dot(xb, vwlo_ref[...], preferred_element_type=jnp.float32) \
        + vblo_ref[...]
    vhi = jnp.dot(xb, vwhi_ref[...], preferred_element_type=jnp.float32) \
        + vbhi_ref[...]
    lo16 = lax.bitcast_convert_type(vlo.astype(jnp.bfloat16), jnp.uint16)
    hi16 = lax.bitcast_convert_type(vhi.astype(jnp.bfloat16), jnp.uint16)
    packed = lo16.astype(jnp.uint32) | (hi16.astype(jnp.uint32) << 16)
    vp_ref[...] = lax.bitcast_convert_type(packed, jnp.int32)


def _stage_a(x2, window_w, window_b, window_gamma, offset_w, offset_b,
             offset_gamma, kernel_w, kernel_b, kernel_gamma,
             vw_lo, vb_lo, vw_hi, vb_hi):
    nblk = L // LB
    full = lambda shape: pl.BlockSpec(shape, lambda i: tuple(0 for _ in shape))
    return pl.pallas_call(
        _stage_a_body,
        grid=(nblk,),
        in_specs=[
            pl.BlockSpec((LB, C), lambda i: (i, 0)),
            full((C, H)), full((H,)), full((H,)),
            full((C, H)), full((H,)), full((H,)),
            full((C, H * K)), full((H * K,)), full((H * K,)),
            full((C, C // 2)), full((C // 2,)),
            full((C, C // 2)), full((C // 2,)),
        ],
        out_specs=[
            pl.BlockSpec((H, LB), lambda i: (0, i)),
            pl.BlockSpec((H, LB), lambda i: (0, i)),
            pl.BlockSpec((LB, H * K), lambda i: (i, 0)),
            pl.BlockSpec((LB, C // 2), lambda i: (i, 0)),
        ],
        out_shape=[
            jax.ShapeDtypeStruct((H, L), jnp.float32),
            jax.ShapeDtypeStruct((H, L), jnp.float32),
            jax.ShapeDtypeStruct((L, H * K), jnp.float32),
            jax.ShapeDtypeStruct((L, C // 2), jnp.int32),
        ],
    )(x2, window_w, window_b, window_gamma, offset_w, offset_b, offset_gamma,
      kernel_w, kernel_b, kernel_gamma, vw_lo, vb_lo, vw_hi, vb_hi)


def _sc_body(ws_hbm, co_hbm, kw_hbm, v_hbm, out_hbm,
             vrows0, vrows1, ktab0, ktab1, wsv0, wsv1, cov0, cov1,
             a_scr, e_scr, row_scr, out_scr, sem0, sem1):
    nc = 2
    wid = lax.axis_index("s") * nc + lax.axis_index("c")
    iota16 = lax.iota(jnp.int32, 16)
    zeros16 = jnp.zeros((16,), jnp.float32)
    bufs = ((vrows0, ktab0, wsv0, cov0, sem0),
            (vrows1, ktab1, wsv1, cov1, sem1))

    for g in range(4):
        a_scr[0, pl.ds(g * 16, 16)] = zeros16
        a_scr[NO + 1, pl.ds(g * 16, 16)] = zeros16

    def task_params(tl):
        tid = wid * TASKS_PER_W + tl
        h = tid // NTASK_L
        l0 = (tid % NTASK_L) * CHUNK
        r0 = jnp.maximum(0, jnp.minimum(l0 - (HALF + NJ), L - ROWS))
        return h, l0, r0

    def task_copies(tl, buf):
        vrows, ktab, wsv, cov, sem = buf
        h, l0, r0 = task_params(tl)
        return sem, [
            (v_hbm.at[pl.ds(r0, ROWS), pl.ds(h * WPH, WPH)],
             vrows.at[:, pl.ds(0, WPH)]),
            (kw_hbm.at[pl.ds(l0, CHUNK), pl.ds(h * K, K)],
             ktab.at[:, pl.ds(0, K)]),
            (ws_hbm.at[h, pl.ds(l0, CHUNK)], wsv),
            (co_hbm.at[h, pl.ds(l0, CHUNK)], cov),
        ]

    def issue(tl, buf):
        sem, cps = task_copies(tl, buf)
        for s, d in cps:
            pltpu.make_async_copy(s, d, sem).start()

    def drain(tl, buf):
        sem, cps = task_copies(tl, buf)
        for s, d in cps:
            pltpu.make_async_copy(s, d, sem).wait()

    def compute(tl, buf):
        vrows, ktab, wsv, cov, _ = buf
        h, l0, r0 = task_params(tl)

        def group_body(g, carry2):
            lofs = g * 16 + iota16
            lvec_f = (l0 + lofs).astype(jnp.float32)
            c16 = cov[pl.ds(g * 16, 16)]
            w16 = wsv[pl.ds(g * 16, 16)]
            fc_t = c16.astype(jnp.int32)
            fc = jnp.where(c16 < fc_t.astype(jnp.float32), fc_t - 1, fc_t)
            fr = c16 - fc.astype(jnp.float32)
            base = l0 + lofs + fc - HALF
            inv_hw = 1.0 / (w16 * 0.5 + 1e-6)

            def tap_body(tp, denom):
                o_f = tp.astype(jnp.float32) - HALF
                npos = lvec_f + c16 + o_f
                valid = (npos >= 0.0) & (npos < float(L))
                rel = jnp.abs(o_f) * inv_hw
                sg = 1.0 / (1.0 + jnp.exp(5.0 * (rel - 1.0)))
                wm = jnp.where(valid, sg, 0.0)
                kidx = jnp.minimum(rel, 1.0) * float(K - 1)
                kf = jnp.minimum(kidx.astype(jnp.int32), K - 2)
                wcc = kidx - kf.astype(jnp.float32)
                kvf = plsc.load_gather(ktab, [lofs, kf])
                kvc = plsc.load_gather(ktab, [lofs, kf + 1])
                a = (kvf * (1.0 - wcc) + kvc * wcc) * wm
                a_scr[tp + 1, pl.ds(g * 16, 16)] = a
                return denom + a

            denom = lax.fori_loop(0, NO, tap_body, zeros16)
            inv_den = 1.0 / (denom + 1e-6)
            fr1 = (1.0 - fr) * inv_den
            fr2 = fr * inv_den

            def fold_body(j, carry3):
                a_j = a_scr[j + 1, pl.ds(g * 16, 16)]
                a_jm1 = a_scr[j, pl.ds(g * 16, 16)]
                e = fr1 * a_j + fr2 * a_jm1
                rowj = jnp.clip(base + j, 0, L - 1) - r0
                e_scr[j, pl.ds(g * 16, 16)] = e
                row_scr[j, pl.ds(g * 16, 16)] = rowj
                return carry3

            lax.fori_loop(0, NJ, fold_body, 0)
            return carry2

        lax.fori_loop(0, 4, group_body, 0)

        WC = 8
        for g in range(4):
            sl = pl.ds(g * 16, 16)
            for wc in range(WPH // WC):
                def j_body(j, accs, _wc=wc, _sl=sl):
                    evec = e_scr[j, _sl]
                    rvec = row_scr[j, _sl]
                    new = []
                    for ww in range(WC):
                        wcol = jnp.full((16,), _wc * WC + ww, jnp.int32)
                        gv = plsc.load_gather(vrows, [rvec, wcol])
                        vlo, vhi = plsc.unpack(
                            plsc.bitcast(gv, jnp.bfloat16),
                            format=plsc.PackFormat.INTERLEAVED)
                        new.append(accs[2 * ww] + evec * vlo)
                        new.append(accs[2 * ww + 1] + evec * vhi)
                    return tuple(new)

                accs = lax.fori_loop(0, NJ, j_body,
                                     tuple([zeros16] * (2 * WC)))
                for ww in range(WC):
                    w = wc * WC + ww
                    out_scr[w, sl] = accs[2 * ww]
                    out_scr[w + WPH, sl] = accs[2 * ww + 1]

        pltpu.sync_copy(out_scr, out_hbm.at[pl.ds(h * D, D), pl.ds(l0, CHUNK)])

    issue(0, bufs[0])

    def pair_body(tp, carry):
        t_even = 2 * tp
        issue(t_even + 1, bufs[1])
        drain(t_even, bufs[0])
        compute(t_even, bufs[0])
        issue(jnp.minimum(t_even + 2, TASKS_PER_W - 1), bufs[0])
        drain(t_even + 1, bufs[1])
        compute(t_even + 1, bufs[1])
        return carry

    lax.fori_loop(0, TASKS_PER_W // 2, pair_body, 0)
    drain(TASKS_PER_W - 1, bufs[0])


def _stage_b(ws_t, co_t, kw, v_packed):
    mesh = plsc.VectorSubcoreMesh(core_axis_name="c", subcore_axis_name="s",
                                  num_cores=2, num_subcores=16)
    f = functools.partial(
        pl.kernel,
        out_type=jax.ShapeDtypeStruct((C, L), jnp.float32),
        mesh=mesh,
        scratch_types=[
            pltpu.VMEM((ROWS, WPH + 1), jnp.int32),
            pltpu.VMEM((ROWS, WPH + 1), jnp.int32),
            pltpu.VMEM((CHUNK, K + 1), jnp.float32),
            pltpu.VMEM((CHUNK, K + 1), jnp.float32),
            pltpu.VMEM((CHUNK,), jnp.float32),
            pltpu.VMEM((CHUNK,), jnp.float32),
            pltpu.VMEM((CHUNK,), jnp.float32),
            pltpu.VMEM((CHUNK,), jnp.float32),
            pltpu.VMEM((NO + 2, CHUNK), jnp.float32),
            pltpu.VMEM((NJ, CHUNK), jnp.float32),
            pltpu.VMEM((NJ, CHUNK), jnp.int32),
            pltpu.VMEM((D, CHUNK), jnp.float32),
            pltpu.SemaphoreType.DMA,
            pltpu.SemaphoreType.DMA,
        ],
        compiler_params=pltpu.CompilerParams(use_tc_tiling_on_sc=False,
                                             needs_layout_passes=False),
    )(_sc_body)
    return f(ws_t, co_t, kw, v_packed)


def _stage_c_body(opt_ref, ow_ref, out_ref):
    y = lax.dot_general(opt_ref[...], ow_ref[...],
                        dimension_numbers=(((0,), (0,)), ((), ())),
                        preferred_element_type=jnp.float32)
    out_ref[...] = y * (1.0 / (1.0 + jnp.exp(-y)))


def _stage_c(op_t, out_w):
    nblk = L // LB
    return pl.pallas_call(
        _stage_c_body,
        grid=(nblk,),
        in_specs=[
            pl.BlockSpec((C, LB), lambda i: (0, i)),
            pl.BlockSpec((C, C), lambda i: (0, 0)),
        ],
        out_specs=pl.BlockSpec((LB, C), lambda i: (i, 0)),
        out_shape=jax.ShapeDtypeStruct((L, C), jnp.float32),
    )(op_t, out_w)


def kernel(x, window_w, window_b, window_gamma, offset_w, offset_b,
           offset_gamma, kernel_w, kernel_b, kernel_gamma, v_w, v_b, out_w):
    x2 = x[0]
    cols = jnp.arange(C // 2)
    idx_lo = (cols // WPH) * D + (cols % WPH)
    idx_hi = idx_lo + WPH
    vw_lo, vb_lo = v_w[:, idx_lo], v_b[idx_lo]
    vw_hi, vb_hi = v_w[:, idx_hi], v_b[idx_hi]
    ws_t, co_t, kw, v_packed = _stage_a(
        x2, window_w, window_b, window_gamma, offset_w, offset_b,
        offset_gamma, kernel_w, kernel_b, kernel_gamma,
        vw_lo, vb_lo, vw_hi, vb_hi)
    op_t = _stage_b(ws_t, co_t, kw, v_packed)
    out = _stage_c(op_t, out_w)
    return out[None]


# speedup vs baseline: 160.1517x; 1.0007x over previous
import functools
import math

import jax
import jax.numpy as jnp
from jax import lax
from jax.experimental import pallas as pl
from jax.experimental.pallas import tpu as pltpu
from jax.experimental.pallas import tpu_sc as plsc

L = 2048
C = 768
H = 12
K = 64
D = C // H
MAX_WINDOW = float(min(int(math.sqrt(L)), K))
HALF = int(MAX_WINDOW) // 2
MAX_OFFSET = float(int(math.sqrt(L)))
NO = 2 * HALF + 1
NJ = NO + 1
MIN_WINDOW = 1.0

LB = 256
CHUNK = 64
WPH = D // 2
ROWS = 200
NTASK_L = L // CHUNK
NTASKS = NTASK_L * H
NWORKERS = 32
TASKS_PER_W = NTASKS // NWORKERS


def _stage_a_body(x_ref, ww_ref, wb_ref, wg_ref, ow_ref, ob_ref, og_ref,
                  kw_ref, kb_ref, kg_ref, vwlo_ref, vblo_ref, vwhi_ref,
                  vbhi_ref, wst_ref, cot_ref, kt_ref, vp_ref):
    xb = x_ref[...]

    def proj_t(w_ref, b_ref):
        p = lax.dot_general(w_ref[...], xb, (((0,), (1,)), ((), ())),
                            preferred_element_type=jnp.float32)
        return p + b_ref[...][:, None]

    def rms_t(p, g_ref):
        var = jnp.mean(p * p, axis=0, keepdims=True)
        return g_ref[...][:, None] * (p * lax.rsqrt(var + 1e-6))

    wn = rms_t(proj_t(ww_ref, wb_ref), wg_ref)
    wst_ref[...] = MIN_WINDOW + (MAX_WINDOW - MIN_WINDOW) / (1.0 + jnp.exp(-wn))
    on = rms_t(proj_t(ow_ref, ob_ref), og_ref)
    cot_ref[...] = jnp.tanh(on) * MAX_OFFSET

    xb16 = xb.astype(jnp.bfloat16)
    kpre = jnp.dot(xb16, kw_ref[...].astype(jnp.bfloat16),
                   preferred_element_type=jnp.float32) + kb_ref[...]
    kvar = jnp.mean(kpre * kpre, axis=-1, keepdims=True)
    kn = kg_ref[...] * (kpre * lax.rsqrt(kvar + 1e-6))
    kt_ref[...] = kn * (1.0 / (1.0 + jnp.exp(-kn)))

    vlo = jnp.dot(xb16, vwlo_ref[...].astype(jnp.bfloat16),
                  preferred_element_type=jnp.float32) + vblo_ref[...]
    vhi = jnp.dot(xb16, vwhi_ref[...].astype(jnp.bfloat16),
                  preferred_element_type=jnp.float32) + vbhi_ref[...]
    lo16 = lax.bitcast_convert_type(vlo.astype(jnp.bfloat16), jnp.uint16)
    hi16 = lax.bitcast_convert_type(vhi.astype(jnp.bfloat16), jnp.uint16)
    packed = lo16.astype(jnp.uint32) | (hi16.astype(jnp.uint32) << 16)
    vp_ref[...] = lax.bitcast_convert_type(packed, jnp.int32)


def _stage_a(x2, window_w, window_b, window_gamma, offset_w, offset_b,
             offset_gamma, kernel_w, kernel_b, kernel_gamma,
             vw_lo, vb_lo, vw_hi, vb_hi):
    nblk = L // LB
    full = lambda shape: pl.BlockSpec(shape, lambda i: tuple(0 for _ in shape))
    return pl.pallas_call(
        _stage_a_body,
        grid=(nblk,),
        in_specs=[
            pl.BlockSpec((LB, C), lambda i: (i, 0)),
            full((C, H)), full((H,)), full((H,)),
            full((C, H)), full((H,)), full((H,)),
            full((C, H * K)), full((H * K,)), full((H * K,)),
            full((C, C // 2)), full((C // 2,)),
            full((C, C // 2)), full((C // 2,)),
        ],
        out_specs=[
            pl.BlockSpec((H, LB), lambda i: (0, i)),
            pl.BlockSpec((H, LB), lambda i: (0, i)),
            pl.BlockSpec((LB, H * K), lambda i: (i, 0)),
            pl.BlockSpec((LB, C // 2), lambda i: (i, 0)),
        ],
        out_shape=[
            jax.ShapeDtypeStruct((H, L), jnp.float32),
            jax.ShapeDtypeStruct((H, L), jnp.float32),
            jax.ShapeDtypeStruct((L, H * K), jnp.float32),
            jax.ShapeDtypeStruct((L, C // 2), jnp.int32),
        ],
    )(x2, window_w, window_b, window_gamma, offset_w, offset_b, offset_gamma,
      kernel_w, kernel_b, kernel_gamma, vw_lo, vb_lo, vw_hi, vb_hi)


def _sc_body(ws_hbm, co_hbm, kw_hbm, v_hbm, out_hbm,
             vrows0, vrows1, ktab0, ktab1, wsv0, wsv1, cov0, cov1,
             a_scr, e_scr, row_scr, out_scr, sem0, sem1):
    nc = 2
    wid = lax.axis_index("s") * nc + lax.axis_index("c")
    iota16 = lax.iota(jnp.int32, 16)
    zeros16 = jnp.zeros((16,), jnp.float32)
    bufs = ((vrows0, ktab0, wsv0, cov0, sem0),
            (vrows1, ktab1, wsv1, cov1, sem1))

    for g in range(4):
        a_scr[0, pl.ds(g * 16, 16)] = zeros16
        a_scr[NO + 1, pl.ds(g * 16, 16)] = zeros16

    def task_params(tl):
        tid = wid * TASKS_PER_W + tl
        h = tid // NTASK_L
        l0 = (tid % NTASK_L) * CHUNK
        r0 = jnp.maximum(0, jnp.minimum(l0 - (HALF + NJ), L - ROWS))
        return h, l0, r0

    def task_copies(tl, buf):
        vrows, ktab, wsv, cov, sem = buf
        h, l0, r0 = task_params(tl)
        return sem, [
            (v_hbm.at[pl.ds(r0, ROWS), pl.ds(h * WPH, WPH)],
             vrows.at[:, pl.ds(0, WPH)]),
            (kw_hbm.at[pl.ds(l0, CHUNK), pl.ds(h * K, K)],
             ktab.at[:, pl.ds(0, K)]),
            (ws_hbm.at[h, pl.ds(l0, CHUNK)], wsv),
            (co_hbm.at[h, pl.ds(l0, CHUNK)], cov),
        ]

    def issue(tl, buf):
        sem, cps = task_copies(tl, buf)
        for s, d in cps:
            pltpu.make_async_copy(s, d, sem).start()

    def drain(tl, buf):
        sem, cps = task_copies(tl, buf)
        for s, d in cps:
            pltpu.make_async_copy(s, d, sem).wait()

    def compute(tl, buf):
        vrows, ktab, wsv, cov, _ = buf
        h, l0, r0 = task_params(tl)

        def group_body(g, carry2):
            lofs = g * 16 + iota16
            lvec_f = (l0 + lofs).astype(jnp.float32)
            c16 = cov[pl.ds(g * 16, 16)]
            w16 = wsv[pl.ds(g * 16, 16)]
            fc_t = c16.astype(jnp.int32)
            fc = jnp.where(c16 < fc_t.astype(jnp.float32), fc_t - 1, fc_t)
            fr = c16 - fc.astype(jnp.float32)
            base = l0 + lofs + fc - HALF
            inv_hw = 1.0 / (w16 * 0.5 + 1e-6)

            def tap_body(tp, denom):
                o_f = tp.astype(jnp.float32) - HALF
                npos = lvec_f + c16 + o_f
                valid = (npos >= 0.0) & (npos < float(L))
                rel = jnp.abs(o_f) * inv_hw
                sg = 1.0 / (1.0 + jnp.exp(5.0 * (rel - 1.0)))
                wm = jnp.where(valid, sg, 0.0)
                kidx = jnp.minimum(rel, 1.0) * float(K - 1)
                kf = jnp.minimum(kidx.astype(jnp.int32), K - 2)
                wcc = kidx - kf.astype(jnp.float32)
                kvf = plsc.load_gather(ktab, [lofs, kf])
                kvc = plsc.load_gather(ktab, [lofs, kf + 1])
                a = (kvf * (1.0 - wcc) + kvc * wcc) * wm
                a_scr[tp + 1, pl.ds(g * 16, 16)] = a
                return denom + a

            denom = lax.fori_loop(0, NO, tap_body, zeros16)
            inv_den = 1.0 / (denom + 1e-6)
            fr1 = (1.0 - fr) * inv_den
            fr2 = fr * inv_den

            def fold_body(j, carry3):
                a_j = a_scr[j + 1, pl.ds(g * 16, 16)]
                a_jm1 = a_scr[j, pl.ds(g * 16, 16)]
                e = fr1 * a_j + fr2 * a_jm1
                rowj = jnp.clip(base + j, 0, L - 1) - r0
                e_scr[j, pl.ds(g * 16, 16)] = e
                row_scr[j, pl.ds(g * 16, 16)] = rowj
                return carry3

            lax.fori_loop(0, NJ, fold_body, 0)
            return carry2

        lax.fori_loop(0, 4, group_body, 0)

        WC = 8
        for g in range(4):
            sl = pl.ds(g * 16, 16)
            for wc in range(WPH // WC):
                def j_body(j, accs, _wc=wc, _sl=sl):
                    evec = e_scr[j, _sl]
                    rvec = row_scr[j, _sl]
                    new = []
                    for ww in range(WC):
                        wcol = jnp.full((16,), _wc * WC + ww, jnp.int32)
                        gv = plsc.load_gather(vrows, [rvec, wcol])
                        vlo, vhi = plsc.unpack(
                            plsc.bitcast(gv, jnp.bfloat16),
                            format=plsc.PackFormat.INTERLEAVED)
                        new.append(accs[2 * ww] + evec * vlo)
                        new.append(accs[2 * ww + 1] + evec * vhi)
                    return tuple(new)

                accs = lax.fori_loop(0, NJ, j_body,
                                     tuple([zeros16] * (2 * WC)))
                for ww in range(WC):
                    w = wc * WC + ww
                    out_scr[w, sl] = accs[2 * ww]
                    out_scr[w + WPH, sl] = accs[2 * ww + 1]

        pltpu.sync_copy(out_scr, out_hbm.at[pl.ds(h * D, D), pl.ds(l0, CHUNK)])

    issue(0, bufs[0])

    def pair_body(tp, carry):
        t_even = 2 * tp
        issue(t_even + 1, bufs[1])
        drain(t_even, bufs[0])
        compute(t_even, bufs[0])
        issue(jnp.minimum(t_even + 2, TASKS_PER_W - 1), bufs[0])
        drain(t_even + 1, bufs[1])
        compute(t_even + 1, bufs[1])
        return carry

    lax.fori_loop(0, TASKS_PER_W // 2, pair_body, 0)
    drain(TASKS_PER_W - 1, bufs[0])


def _stage_b(ws_t, co_t, kw, v_packed):
    mesh = plsc.VectorSubcoreMesh(core_axis_name="c", subcore_axis_name="s",
                                  num_cores=2, num_subcores=16)
    f = functools.partial(
        pl.kernel,
        out_type=jax.ShapeDtypeStruct((C, L), jnp.float32),
        mesh=mesh,
        scratch_types=[
            pltpu.VMEM((ROWS, WPH + 1), jnp.int32),
            pltpu.VMEM((ROWS, WPH + 1), jnp.int32),
            pltpu.VMEM((CHUNK, K + 1), jnp.float32),
            pltpu.VMEM((CHUNK, K + 1), jnp.float32),
            pltpu.VMEM((CHUNK,), jnp.float32),
            pltpu.VMEM((CHUNK,), jnp.float32),
            pltpu.VMEM((CHUNK,), jnp.float32),
            pltpu.VMEM((CHUNK,), jnp.float32),
            pltpu.VMEM((NO + 2, CHUNK), jnp.float32),
            pltpu.VMEM((NJ, CHUNK), jnp.float32),
            pltpu.VMEM((NJ, CHUNK), jnp.int32),
            pltpu.VMEM((D, CHUNK), jnp.float32),
            pltpu.SemaphoreType.DMA,
            pltpu.SemaphoreType.DMA,
        ],
        compiler_params=pltpu.CompilerParams(use_tc_tiling_on_sc=False,
                                             needs_layout_passes=False),
    )(_sc_body)
    return f(ws_t, co_t, kw, v_packed)


def _stage_c_body(opt_ref, ow_ref, out_ref):
    y = lax.dot_general(opt_ref[...].astype(jnp.bfloat16),
                        ow_ref[...].astype(jnp.bfloat16),
                        dimension_numbers=(((0,), (0,)), ((), ())),
                        preferred_element_type=jnp.float32)
    out_ref[...] = y * (1.0 / (1.0 + jnp.exp(-y)))


def _stage_c(op_t, out_w):
    nblk = L // LB
    return pl.pallas_call(
        _stage_c_body,
        grid=(nblk,),
        in_specs=[
            pl.BlockSpec((C, LB), lambda i: (0, i)),
            pl.BlockSpec((C, C), lambda i: (0, 0)),
        ],
        out_specs=pl.BlockSpec((LB, C), lambda i: (i, 0)),
        out_shape=jax.ShapeDtypeStruct((L, C), jnp.float32),
    )(op_t, out_w)


def kernel(x, window_w, window_b, window_gamma, offset_w, offset_b,
           offset_gamma, kernel_w, kernel_b, kernel_gamma, v_w, v_b, out_w):
    x2 = x[0]
    cols = jnp.arange(C // 2)
    idx_lo = (cols // WPH) * D + (cols % WPH)
    idx_hi = idx_lo + WPH
    vw_lo, vb_lo = v_w[:, idx_lo], v_b[idx_lo]
    vw_hi, vb_hi = v_w[:, idx_hi], v_b[idx_hi]
    ws_t, co_t, kw, v_packed = _stage_a(
        x2, window_w, window_b, window_gamma, offset_w, offset_b,
        offset_gamma, kernel_w, kernel_b, kernel_gamma,
        vw_lo, vb_lo, vw_hi, vb_hi)
    op_t = _stage_b(ws_t, co_t, kw, v_packed)
    out = _stage_c(op_t, out_w)
    return out[None]


# speedup vs baseline: 170.0285x; 1.0617x over previous
import functools
import math

import jax
import jax.numpy as jnp
from jax import lax
from jax.experimental import pallas as pl
from jax.experimental.pallas import tpu as pltpu
from jax.experimental.pallas import tpu_sc as plsc

L = 2048
C = 768
H = 12
K = 64
D = C // H
MAX_WINDOW = float(min(int(math.sqrt(L)), K))
HALF = int(MAX_WINDOW) // 2
MAX_OFFSET = float(int(math.sqrt(L)))
NO = 2 * HALF + 1
NJ = NO + 1
MIN_WINDOW = 1.0

LB = 256
CHUNK = 64
WPH = D // 2
ROWS = 200
NTASK_L = L // CHUNK
NTASKS = NTASK_L * H
NWORKERS = 32
TASKS_PER_W = NTASKS // NWORKERS


def _stage_a_body(x_ref, ww_ref, wb_ref, wg_ref, ow_ref, ob_ref, og_ref,
                  kw_ref, kb_ref, kg_ref, vwlo_ref, vblo_ref, vwhi_ref,
                  vbhi_ref, wst_ref, cot_ref, kt_ref, vp_ref):
    xb = x_ref[...]

    def proj_t(w_ref, b_ref):
        p = lax.dot_general(w_ref[...], xb, (((0,), (1,)), ((), ())),
                            preferred_element_type=jnp.float32)
        return p + b_ref[...][:, None]

    def rms_t(p, g_ref):
        var = jnp.mean(p * p, axis=0, keepdims=True)
        return g_ref[...][:, None] * (p * lax.rsqrt(var + 1e-6))

    wn = rms_t(proj_t(ww_ref, wb_ref), wg_ref)
    wst_ref[...] = MIN_WINDOW + (MAX_WINDOW - MIN_WINDOW) / (1.0 + jnp.exp(-wn))
    on = rms_t(proj_t(ow_ref, ob_ref), og_ref)
    cot_ref[...] = jnp.tanh(on) * MAX_OFFSET

    xb16 = xb.astype(jnp.bfloat16)
    kpre = jnp.dot(xb16, kw_ref[...].astype(jnp.bfloat16),
                   preferred_element_type=jnp.float32) + kb_ref[...]
    kvar = jnp.mean(kpre * kpre, axis=-1, keepdims=True)
    kn = kg_ref[...] * (kpre * lax.rsqrt(kvar + 1e-6))
    kt_ref[...] = kn * (1.0 / (1.0 + jnp.exp(-kn)))

    vlo = jnp.dot(xb16, vwlo_ref[...].astype(jnp.bfloat16),
                  preferred_element_type=jnp.float32) + vblo_ref[...]
    vhi = jnp.dot(xb16, vwhi_ref[...].astype(jnp.bfloat16),
                  preferred_element_type=jnp.float32) + vbhi_ref[...]
    lo16 = lax.bitcast_convert_type(vlo.astype(jnp.bfloat16), jnp.uint16)
    hi16 = lax.bitcast_convert_type(vhi.astype(jnp.bfloat16), jnp.uint16)
    packed = lo16.astype(jnp.uint32) | (hi16.astype(jnp.uint32) << 16)
    vp_ref[...] = lax.bitcast_convert_type(packed, jnp.int32)


def _stage_a(x2, window_w, window_b, window_gamma, offset_w, offset_b,
             offset_gamma, kernel_w, kernel_b, kernel_gamma,
             vw_lo, vb_lo, vw_hi, vb_hi):
    nblk = L // LB
    full = lambda shape: pl.BlockSpec(shape, lambda i: tuple(0 for _ in shape))
    return pl.pallas_call(
        _stage_a_body,
        grid=(nblk,),
        in_specs=[
            pl.BlockSpec((LB, C), lambda i: (i, 0)),
            full((C, H)), full((H,)), full((H,)),
            full((C, H)), full((H,)), full((H,)),
            full((C, H * K)), full((H * K,)), full((H * K,)),
            full((C, C // 2)), full((C // 2,)),
            full((C, C // 2)), full((C // 2,)),
        ],
        out_specs=[
            pl.BlockSpec((H, LB), lambda i: (0, i)),
            pl.BlockSpec((H, LB), lambda i: (0, i)),
            pl.BlockSpec((LB, H * K), lambda i: (i, 0)),
            pl.BlockSpec((LB, C // 2), lambda i: (i, 0)),
        ],
        out_shape=[
            jax.ShapeDtypeStruct((H, L), jnp.float32),
            jax.ShapeDtypeStruct((H, L), jnp.float32),
            jax.ShapeDtypeStruct((L, H * K), jnp.float32),
            jax.ShapeDtypeStruct((L, C // 2), jnp.int32),
        ],
    )(x2, window_w, window_b, window_gamma, offset_w, offset_b, offset_gamma,
      kernel_w, kernel_b, kernel_gamma, vw_lo, vb_lo, vw_hi, vb_hi)


def _sc_body(ws_hbm, co_hbm, kw_hbm, v_hbm, out_hbm,
             vrows0, vrows1, ktab0, ktab1, wsv0, wsv1, cov0, cov1,
             a_scr, e_scr, row_scr, out_scr, sem0, sem1):
    nc = 2
    wid = lax.axis_index("s") * nc + lax.axis_index("c")
    iota16 = lax.iota(jnp.int32, 16)
    zeros16 = jnp.zeros((16,), jnp.float32)
    bufs = ((vrows0, ktab0, wsv0, cov0, sem0),
            (vrows1, ktab1, wsv1, cov1, sem1))

    for g in range(4):
        a_scr[0, pl.ds(g * 16, 16)] = zeros16
        a_scr[NO + 1, pl.ds(g * 16, 16)] = zeros16

    def task_params(tl):
        tid = wid * TASKS_PER_W + tl
        h = tid // NTASK_L
        l0 = (tid % NTASK_L) * CHUNK
        r0 = jnp.maximum(0, jnp.minimum(l0 - (HALF + NJ), L - ROWS))
        return h, l0, r0

    def task_copies(tl, buf):
        vrows, ktab, wsv, cov, sem = buf
        h, l0, r0 = task_params(tl)
        return sem, [
            (v_hbm.at[pl.ds(r0, ROWS), pl.ds(h * WPH, WPH)],
             vrows.at[:, pl.ds(0, WPH)]),
            (kw_hbm.at[pl.ds(l0, CHUNK), pl.ds(h * K, K)],
             ktab.at[:, pl.ds(0, K)]),
            (ws_hbm.at[h, pl.ds(l0, CHUNK)], wsv),
            (co_hbm.at[h, pl.ds(l0, CHUNK)], cov),
        ]

    def issue(tl, buf):
        sem, cps = task_copies(tl, buf)
        for s, d in cps:
            pltpu.make_async_copy(s, d, sem).start()

    def drain(tl, buf):
        sem, cps = task_copies(tl, buf)
        for s, d in cps:
            pltpu.make_async_copy(s, d, sem).wait()

    def compute(tl, buf):
        vrows, ktab, wsv, cov, _ = buf
        h, l0, r0 = task_params(tl)

        def group_body(g, carry2):
            lofs = g * 16 + iota16
            lvec_f = (l0 + lofs).astype(jnp.float32)
            c16 = cov[pl.ds(g * 16, 16)]
            w16 = wsv[pl.ds(g * 16, 16)]
            fc_t = c16.astype(jnp.int32)
            fc = jnp.where(c16 < fc_t.astype(jnp.float32), fc_t - 1, fc_t)
            fr = c16 - fc.astype(jnp.float32)
            base = l0 + lofs + fc - HALF
            inv_hw = 1.0 / (w16 * 0.5 + 1e-6)

            def tap_body(tp, denom):
                o_f = tp.astype(jnp.float32) - HALF
                npos = lvec_f + c16 + o_f
                valid = (npos >= 0.0) & (npos < float(L))
                rel = jnp.abs(o_f) * inv_hw
                sg = 1.0 / (1.0 + jnp.exp(5.0 * (rel - 1.0)))
                wm = jnp.where(valid, sg, 0.0)
                kidx = jnp.minimum(rel, 1.0) * float(K - 1)
                kf = jnp.minimum(kidx.astype(jnp.int32), K - 2)
                wcc = kidx - kf.astype(jnp.float32)
                kvf = plsc.load_gather(ktab, [lofs, kf])
                kvc = plsc.load_gather(ktab, [lofs, kf + 1])
                a = (kvf * (1.0 - wcc) + kvc * wcc) * wm
                a_scr[tp + 1, pl.ds(g * 16, 16)] = a
                return denom + a

            denom = lax.fori_loop(0, NO, tap_body, zeros16)
            inv_den = 1.0 / (denom + 1e-6)
            fr1 = (1.0 - fr) * inv_den
            fr2 = fr * inv_den

            def fold_body(j, carry3):
                a_j = a_scr[j + 1, pl.ds(g * 16, 16)]
                a_jm1 = a_scr[j, pl.ds(g * 16, 16)]
                e = fr1 * a_j + fr2 * a_jm1
                rowj = jnp.clip(base + j, 0, L - 1) - r0
                e_scr[j, pl.ds(g * 16, 16)] = e
                row_scr[j, pl.ds(g * 16, 16)] = rowj
                return carry3

            lax.fori_loop(0, NJ, fold_body, 0)
            return carry2

        lax.fori_loop(0, 4, group_body, 0)

        WC = 8
        for g in range(4):
            sl = pl.ds(g * 16, 16)
            for wc in range(WPH // WC):
                def j_body(j, accs, _wc=wc, _sl=sl):
                    evec = e_scr[j, _sl]
                    rvec = row_scr[j, _sl]
                    ebf = plsc.pack(evec, evec,
                                    format=plsc.PackFormat.INTERLEAVED)
                    new = []
                    for ww in range(WC):
                        wcol = jnp.full((16,), _wc * WC + ww, jnp.int32)
                        gv = plsc.load_gather(vrows, [rvec, wcol])
                        prod = plsc.bitcast(gv, jnp.bfloat16) * ebf
                        plo, phi = plsc.unpack(
                            prod, format=plsc.PackFormat.INTERLEAVED)
                        new.append(accs[2 * ww] + plo)
                        new.append(accs[2 * ww + 1] + phi)
                    return tuple(new)

                accs = lax.fori_loop(0, NJ, j_body,
                                     tuple([zeros16] * (2 * WC)))
                for ww in range(WC):
                    w = wc * WC + ww
                    out_scr[w, sl] = accs[2 * ww]
                    out_scr[w + WPH, sl] = accs[2 * ww + 1]

        pltpu.sync_copy(out_scr, out_hbm.at[pl.ds(h * D, D), pl.ds(l0, CHUNK)])

    issue(0, bufs[0])

    def pair_body(tp, carry):
        t_even = 2 * tp
        issue(t_even + 1, bufs[1])
        drain(t_even, bufs[0])
        compute(t_even, bufs[0])
        issue(jnp.minimum(t_even + 2, TASKS_PER_W - 1), bufs[0])
        drain(t_even + 1, bufs[1])
        compute(t_even + 1, bufs[1])
        return carry

    lax.fori_loop(0, TASKS_PER_W // 2, pair_body, 0)
    drain(TASKS_PER_W - 1, bufs[0])


def _stage_b(ws_t, co_t, kw, v_packed):
    mesh = plsc.VectorSubcoreMesh(core_axis_name="c", subcore_axis_name="s",
                                  num_cores=2, num_subcores=16)
    f = functools.partial(
        pl.kernel,
        out_type=jax.ShapeDtypeStruct((C, L), jnp.float32),
        mesh=mesh,
        scratch_types=[
            pltpu.VMEM((ROWS, WPH + 1), jnp.int32),
            pltpu.VMEM((ROWS, WPH + 1), jnp.int32),
            pltpu.VMEM((CHUNK, K + 1), jnp.float32),
            pltpu.VMEM((CHUNK, K + 1), jnp.float32),
            pltpu.VMEM((CHUNK,), jnp.float32),
            pltpu.VMEM((CHUNK,), jnp.float32),
            pltpu.VMEM((CHUNK,), jnp.float32),
            pltpu.VMEM((CHUNK,), jnp.float32),
            pltpu.VMEM((NO + 2, CHUNK), jnp.float32),
            pltpu.VMEM((NJ, CHUNK), jnp.float32),
            pltpu.VMEM((NJ, CHUNK), jnp.int32),
            pltpu.VMEM((D, CHUNK), jnp.float32),
            pltpu.SemaphoreType.DMA,
            pltpu.SemaphoreType.DMA,
        ],
        compiler_params=pltpu.CompilerParams(use_tc_tiling_on_sc=False,
                                             needs_layout_passes=False),
    )(_sc_body)
    return f(ws_t, co_t, kw, v_packed)


def _stage_c_body(opt_ref, ow_ref, out_ref):
    y = lax.dot_general(opt_ref[...].astype(jnp.bfloat16),
                        ow_ref[...].astype(jnp.bfloat16),
                        dimension_numbers=(((0,), (0,)), ((), ())),
                        preferred_element_type=jnp.float32)
    out_ref[...] = y * (1.0 / (1.0 + jnp.exp(-y)))


def _stage_c(op_t, out_w):
    nblk = L // LB
    return pl.pallas_call(
        _stage_c_body,
        grid=(nblk,),
        in_specs=[
            pl.BlockSpec((C, LB), lambda i: (0, i)),
            pl.BlockSpec((C, C), lambda i: (0, 0)),
        ],
        out_specs=pl.BlockSpec((LB, C), lambda i: (i, 0)),
        out_shape=jax.ShapeDtypeStruct((L, C), jnp.float32),
    )(op_t, out_w)


def kernel(x, window_w, window_b, window_gamma, offset_w, offset_b,
           offset_gamma, kernel_w, kernel_b, kernel_gamma, v_w, v_b, out_w):
    x2 = x[0]
    cols = jnp.arange(C // 2)
    idx_lo = (cols // WPH) * D + (cols % WPH)
    idx_hi = idx_lo + WPH
    vw_lo, vb_lo = v_w[:, idx_lo], v_b[idx_lo]
    vw_hi, vb_hi = v_w[:, idx_hi], v_b[idx_hi]
    ws_t, co_t, kw, v_packed = _stage_a(
        x2, window_w, window_b, window_gamma, offset_w, offset_b,
        offset_gamma, kernel_w, kernel_b, kernel_gamma,
        vw_lo, vb_lo, vw_hi, vb_hi)
    op_t = _stage_b(ws_t, co_t, kw, v_packed)
    out = _stage_c(op_t, out_w)
    return out[None]
